# baseline (speedup 1.0000x reference)
import jax
import jax.numpy as jnp
from jax.experimental import pallas as pl
from jax.experimental.pallas import tpu as pltpu


_BS = 8


def _pad_body(in_ref, out_hbm, scratch, s_bot, s_right, s_corner,
              sem_bulk, sem_b, sem_r, sem_c):
    i = pl.program_id(0)
    x = in_ref[...]
    rolled = jnp.roll(x, shift=(1, 1), axis=(1, 2))
    scratch[...] = rolled
    s_bot[...] = rolled[:, 0:2, :]
    s_right[...] = rolled[:, :, 0:2]
    s_corner[...] = rolled[:, 0:2, 0:2]

    base = i * _BS
    bulk = pltpu.make_async_copy(
        scratch,
        out_hbm.at[pl.ds(base, _BS), pl.ds(0, 512), pl.ds(0, 512)],
        sem_bulk,
    )
    bot = pltpu.make_async_copy(
        s_bot,
        out_hbm.at[pl.ds(base, _BS), pl.ds(512, 2), pl.ds(0, 512)],
        sem_b,
    )
    right = pltpu.make_async_copy(
        s_right,
        out_hbm.at[pl.ds(base, _BS), pl.ds(0, 512), pl.ds(512, 2)],
        sem_r,
    )
    corner = pltpu.make_async_copy(
        s_corner,
        out_hbm.at[pl.ds(base, _BS), pl.ds(512, 2), pl.ds(512, 2)],
        sem_c,
    )
    bulk.start()
    bot.start()
    right.start()
    corner.start()
    bulk.wait()
    bot.wait()
    right.wait()
    corner.wait()


def kernel(inputs):
    b, h, w = inputs.shape
    return pl.pallas_call(
        _pad_body,
        grid=(b // _BS,),
        in_specs=[pl.BlockSpec((_BS, h, w), lambda i: (i, 0, 0))],
        out_specs=pl.BlockSpec(memory_space=pl.ANY),
        out_shape=jax.ShapeDtypeStruct((b, h + 2, w + 2), inputs.dtype),
        scratch_shapes=[
            pltpu.VMEM((_BS, h, w), inputs.dtype),
            pltpu.VMEM((_BS, 2, w), inputs.dtype),
            pltpu.VMEM((_BS, h, 2), inputs.dtype),
            pltpu.VMEM((_BS, 2, 2), inputs.dtype),
            pltpu.SemaphoreType.DMA,
            pltpu.SemaphoreType.DMA,
            pltpu.SemaphoreType.DMA,
            pltpu.SemaphoreType.DMA,
        ],
    )(inputs)
